# baseline (speedup 1.0000x reference)
import functools

import jax
import jax.numpy as jnp
from jax import lax
from jax.experimental import pallas as pl
from jax.experimental.pallas import tpu as pltpu
from jax.experimental.pallas import tpu_sc as plsc

N = 10000
E = 320000
D_IN = 128
H = 128
C = 40
C_PAD = 128
BN_SCALE = 1.0 / (1.0 + 1e-5) ** 0.5

NC = 2
NS = 16
NW = NC * NS
LK = 16

K = 128
QC = 80
EW = K * QC
E_PAD = EW * NW
N_PAD = 10240
RPT = N_PAD // NS
ZR = 128

RB = 1024


_SC_MESH = plsc.VectorSubcoreMesh(core_axis_name="c", subcore_axis_name="s")
_SC_PARAMS = pltpu.CompilerParams(needs_layout_passes=False)


@functools.partial(
    pl.kernel,
    out_type=jax.ShapeDtypeStruct((NC, 2, N_PAD), jnp.float32),
    mesh=_SC_MESH,
    compiler_params=_SC_PARAMS,
    scratch_types=[
        pltpu.VMEM((QC, K), jnp.int32),
        pltpu.VMEM((QC, K), jnp.int32),
        pltpu.VMEM((N_PAD,), jnp.float32),
        pltpu.VMEM((N_PAD,), jnp.float32),
        pltpu.VMEM((RPT,), jnp.float32),
        pltpu.VMEM((RPT,), jnp.float32),
        pltpu.MemorySpace.VMEM_SHARED((2, NS, N_PAD), jnp.float32),
    ],
)
def _deg_kernel(srcq, dstq, out, src_v, dst_v, deg_s, deg_d, tmp_v, acc_v, sh):
    c = lax.axis_index("c")
    s = lax.axis_index("s")
    w = s * NC + c

    pltpu.sync_copy(srcq.at[w], src_v)
    pltpu.sync_copy(dstq.at[w], dst_v)

    zeros = jnp.zeros((LK,), jnp.float32)

    @pl.loop(0, N_PAD // LK)
    def _(i):
        deg_s[pl.ds(i * LK, LK)] = zeros
        deg_d[pl.ds(i * LK, LK)] = zeros

    ones = jnp.ones((LK,), jnp.float32)

    @pl.loop(0, QC)
    def _(q):
        for g in range(K // LK):
            idx_s = src_v[q, pl.ds(g * LK, LK)]
            plsc.addupdate_scatter(deg_s, [idx_s], ones)
            idx_d = dst_v[q, pl.ds(g * LK, LK)]
            plsc.addupdate_scatter(deg_d, [idx_d], ones)

    pltpu.sync_copy(deg_s, sh.at[0, s])
    pltpu.sync_copy(deg_d, sh.at[1, s])
    plsc.subcore_barrier()

    for d in range(2):
        pltpu.sync_copy(sh.at[d, 0, pl.ds(s * RPT, RPT)], acc_v)
        for p in range(1, NS):
            pltpu.sync_copy(sh.at[d, p, pl.ds(s * RPT, RPT)], tmp_v)
            for j in range(RPT // LK):
                sl = pl.ds(j * LK, LK)
                acc_v[sl] = acc_v[sl] + tmp_v[sl]
        pltpu.sync_copy(acc_v, out.at[c, d, pl.ds(s * RPT, RPT)])


KA = 80
QA = EW // KA
NP = 4
QP = QA // NP
NB = 4


def _make_agg(W):
    @functools.partial(
        pl.kernel,
        out_type=jax.ShapeDtypeStruct((NC, N_PAD, W), jnp.float32),
        mesh=_SC_MESH,
        compiler_params=_SC_PARAMS,
        scratch_types=[
            pltpu.VMEM((QP, KA), jnp.int32),
            pltpu.VMEM((QP, KA), jnp.int32),
            [pltpu.VMEM((KA, W), jnp.float32)] * NB,
            [pltpu.SemaphoreType.DMA] * NB,
            [pltpu.SemaphoreType.DMA] * NB,
            pltpu.MemorySpace.VMEM_SHARED((N_PAD, W), jnp.float32),
        ],
    )
    def _agg(m_hbm, srcq, dstq, out, src_v, dst_v, rows, gsem, ssem, acc):
        c = lax.axis_index("c")
        s = lax.axis_index("s")
        w = s * NC + c

        zeros = jnp.zeros((LK,), jnp.float32)

        @pl.loop(0, KA)
        def _(i):
            for j in range(W // LK):
                rows[0][i, pl.ds(j * LK, LK)] = zeros

        for r in range(RPT // KA):
            pltpu.sync_copy(rows[0], acc.at[pl.ds(s * RPT + r * KA, KA)])

        plsc.subcore_barrier()

        def gather(q, b):
            pltpu.async_copy(m_hbm.at[src_v.at[q]], rows[b], gsem[b])

        def gather_wait(q, b):
            pltpu.make_async_copy(m_hbm.at[src_v.at[q]], rows[b], gsem[b]).wait()

        def scatter(q, b):
            pltpu.async_copy(rows[b], acc.at[dst_v.at[q]], ssem[b], add=True)

        def scatter_wait(q, b):
            pltpu.make_async_copy(rows[b], acc.at[dst_v.at[q]], ssem[b]).wait()

        for p in range(NP):
            pltpu.sync_copy(srcq.at[w, pl.ds(p * QP, QP)], src_v)
            pltpu.sync_copy(dstq.at[w, pl.ds(p * QP, QP)], dst_v)

            for b in range(NB - 1):
                gather(b, b)

            @pl.loop(0, QP, step=NB)
            def _(q0):
                for b in range(NB):
                    q = q0 + b
                    gather_wait(q, b)
                    scatter(q, b)
                    bp = (b - 1) % NB

                    @pl.when(q + NB - 1 < QP)
                    def _():
                        @pl.when(q >= 1)
                        def _():
                            scatter_wait(q - 1, bp)

                        gather(q + NB - 1, bp)

            for b in range(NB):
                scatter_wait(QP - NB + b, b)

        plsc.subcore_barrier()

        for r in range(RPT // KA):
            sl = pl.ds(s * RPT + r * KA, KA)
            pltpu.sync_copy(acc.at[sl], out.at[c, sl])

    return _agg


_agg_h = _make_agg(H)
_agg_c = _agg_h


def _tc0_body(feat_ref, w_ref, l_ref, od0, od1, m_ref, s_ref):
    x = feat_ref[...]
    ns = lax.rsqrt(jnp.maximum(od0[...] + od1[...], 1.0))
    m_ref[...] = jnp.dot(x * ns, w_ref[...], preferred_element_type=jnp.float32)
    s_ref[...] = jnp.dot(x, l_ref[...], preferred_element_type=jnp.float32)


def _col(shape):
    return pl.BlockSpec(shape, lambda i: (0,) * len(shape))


def _rowblk(w):
    return pl.BlockSpec((RB, w), lambda i: (i, 0))


_DEG_SPEC = pl.BlockSpec((RB, 1), lambda i: (i, 0))


_tc0 = pl.pallas_call(
    _tc0_body,
    grid=(N_PAD // RB,),
    in_specs=[_rowblk(D_IN), _col((D_IN, H)), _col((D_IN, H)), _DEG_SPEC, _DEG_SPEC],
    out_specs=[_rowblk(H), _rowblk(H)],
    out_shape=[
        jax.ShapeDtypeStruct((N_PAD, H), jnp.float32),
        jax.ShapeDtypeStruct((N_PAD, H), jnp.float32),
    ],
)


def _make_tc_mid(wout):
    def body(aggp, sp_ref, id0, id1, od0, od1, g_ref, be_ref, w_ref, l_ref,
             m_ref, s_ref):
        a = aggp[0] + aggp[1]
        nd = lax.rsqrt(jnp.maximum(id0[...] + id1[...], 1.0))
        h = a * nd + sp_ref[...]
        h = h * (g_ref[...] * BN_SCALE) + be_ref[...]
        h = jnp.maximum(h, 0.0)
        ns = lax.rsqrt(jnp.maximum(od0[...] + od1[...], 1.0))
        m_ref[...] = jnp.dot(h * ns, w_ref[...], preferred_element_type=jnp.float32)
        s_ref[...] = jnp.dot(h, l_ref[...], preferred_element_type=jnp.float32)

    return pl.pallas_call(
        body,
        grid=(N_PAD // RB,),
        in_specs=[
            pl.BlockSpec((NC, RB, H), lambda i: (0, i, 0)),
            _rowblk(H), _DEG_SPEC, _DEG_SPEC, _DEG_SPEC, _DEG_SPEC,
            _col((1, H)), _col((1, H)), _col((D_IN, wout)), _col((D_IN, wout)),
        ],
        out_specs=[_rowblk(wout), _rowblk(wout)],
        out_shape=[
            jax.ShapeDtypeStruct((N_PAD, wout), jnp.float32),
            jax.ShapeDtypeStruct((N_PAD, wout), jnp.float32),
        ],
    )


_tc1 = _make_tc_mid(H)
_tc2 = _make_tc_mid(C_PAD)


def _tc3_body(aggp, sp_ref, id0, id1, b_ref, out_ref):
    a = aggp[0] + aggp[1]
    nd = lax.rsqrt(jnp.maximum(id0[...] + id1[...], 1.0))
    out_ref[...] = a * nd + b_ref[...] + sp_ref[...]


_tc3 = pl.pallas_call(
    _tc3_body,
    grid=(N_PAD // RB,),
    in_specs=[
        pl.BlockSpec((NC, RB, C_PAD), lambda i: (0, i, 0)),
        _rowblk(C_PAD), _DEG_SPEC, _DEG_SPEC, _col((1, C_PAD)),
    ],
    out_specs=pl.BlockSpec((RB, C_PAD), lambda i: (i, 0)),
    out_shape=jax.ShapeDtypeStruct((N, C), jnp.float32),
)


def kernel(feat, edge_index, order_attn, W0, W1, W2, b2, L0, L1, L2,
           g0, be0, g1, be1):
    del order_attn
    fill = N + jax.lax.iota(jnp.int32, E_PAD - E) % (N_PAD - N)
    src_pad = jnp.concatenate([edge_index[0], fill])
    dst_pad = jnp.concatenate([edge_index[1], fill])
    srcq = src_pad.reshape(NW, QA, KA)
    dstq = dst_pad.reshape(NW, QA, KA)

    degs = _deg_kernel(src_pad.reshape(NW, QC, K), dst_pad.reshape(NW, QC, K))
    od0 = degs[0, 0].reshape(N_PAD, 1)
    od1 = degs[1, 0].reshape(N_PAD, 1)
    id0 = degs[0, 1].reshape(N_PAD, 1)
    id1 = degs[1, 1].reshape(N_PAD, 1)

    W2p = jnp.pad(W2, ((0, 0), (0, C_PAD - C)))
    L2p = jnp.pad(L2, ((0, 0), (0, C_PAD - C)))
    b2p = jnp.pad(b2, (0, C_PAD - C)).reshape(1, C_PAD)
    g0r, be0r = g0.reshape(1, H), be0.reshape(1, H)
    g1r, be1r = g1.reshape(1, H), be1.reshape(1, H)

    m0, s0 = _tc0(feat, W0, L0, od0, od1)
    a0 = _agg_h(m0, srcq, dstq)
    m1, s1 = _tc1(a0, s0, id0, id1, od0, od1, g0r, be0r, W1, L1)
    a1 = _agg_h(m1, srcq, dstq)
    m2, s2 = _tc2(a1, s1, id0, id1, od0, od1, g1r, be1r, W2p, L2p)
    a2 = _agg_c(m2, srcq, dstq)
    return _tc3(a2, s2, id0, id1, b2p)


# speedup vs baseline: 1.0044x; 1.0044x over previous
import functools

import jax
import jax.numpy as jnp
from jax import lax
from jax.experimental import pallas as pl
from jax.experimental.pallas import tpu as pltpu
from jax.experimental.pallas import tpu_sc as plsc

N = 10000
E = 320000
D_IN = 128
H = 128
C = 40
C_PAD = 128
BN_SCALE = 1.0 / (1.0 + 1e-5) ** 0.5

NC = 2
NS = 16
NW = NC * NS
LK = 16

K = 128
QC = 80
EW = K * QC
E_PAD = EW * NW
N_PAD = 10240
RPT = N_PAD // NS
ZR = 128

RB = 1024


_SC_MESH = plsc.VectorSubcoreMesh(core_axis_name="c", subcore_axis_name="s")
_SC_PARAMS = pltpu.CompilerParams(needs_layout_passes=False)


@functools.partial(
    pl.kernel,
    out_type=jax.ShapeDtypeStruct((NC, 2, N_PAD), jnp.float32),
    mesh=_SC_MESH,
    compiler_params=_SC_PARAMS,
    scratch_types=[
        pltpu.VMEM((QC, K), jnp.int32),
        pltpu.VMEM((QC, K), jnp.int32),
        pltpu.VMEM((N_PAD,), jnp.float32),
        pltpu.VMEM((N_PAD,), jnp.float32),
        pltpu.VMEM((RPT,), jnp.float32),
        pltpu.VMEM((RPT,), jnp.float32),
        pltpu.MemorySpace.VMEM_SHARED((2, NS, N_PAD), jnp.float32),
    ],
)
def _deg_kernel(srcq, dstq, out, src_v, dst_v, deg_s, deg_d, tmp_v, acc_v, sh):
    c = lax.axis_index("c")
    s = lax.axis_index("s")
    w = s * NC + c

    pltpu.sync_copy(srcq.at[w], src_v)
    pltpu.sync_copy(dstq.at[w], dst_v)

    zeros = jnp.zeros((LK,), jnp.float32)

    @pl.loop(0, N_PAD // LK)
    def _(i):
        deg_s[pl.ds(i * LK, LK)] = zeros
        deg_d[pl.ds(i * LK, LK)] = zeros

    ones = jnp.ones((LK,), jnp.float32)

    @pl.loop(0, QC)
    def _(q):
        for g in range(K // LK):
            idx_s = src_v[q, pl.ds(g * LK, LK)]
            plsc.addupdate_scatter(deg_s, [idx_s], ones)
            idx_d = dst_v[q, pl.ds(g * LK, LK)]
            plsc.addupdate_scatter(deg_d, [idx_d], ones)

    pltpu.sync_copy(deg_s, sh.at[0, s])
    pltpu.sync_copy(deg_d, sh.at[1, s])
    plsc.subcore_barrier()

    for d in range(2):
        pltpu.sync_copy(sh.at[d, 0, pl.ds(s * RPT, RPT)], acc_v)
        for p in range(1, NS):
            pltpu.sync_copy(sh.at[d, p, pl.ds(s * RPT, RPT)], tmp_v)
            for j in range(RPT // LK):
                sl = pl.ds(j * LK, LK)
                acc_v[sl] = acc_v[sl] + tmp_v[sl]
        pltpu.sync_copy(acc_v, out.at[c, d, pl.ds(s * RPT, RPT)])


NP = 2
QP = QC // NP


def _make_agg(W):
    @functools.partial(
        pl.kernel,
        out_type=jax.ShapeDtypeStruct((NC, N_PAD, W), jnp.float32),
        mesh=_SC_MESH,
        compiler_params=_SC_PARAMS,
        scratch_types=[
            pltpu.VMEM((QP, K), jnp.int32),
            pltpu.VMEM((QP, K), jnp.int32),
            pltpu.VMEM((K, W), jnp.float32),
            pltpu.VMEM((K, W), jnp.float32),
            pltpu.MemorySpace.VMEM_SHARED((N_PAD, W), jnp.float32),
            pltpu.SemaphoreType.DMA,
            pltpu.SemaphoreType.DMA,
        ],
    )
    def _agg(m_hbm, srcq, dstq, out, src_v, dst_v, rows0, rows1, acc, s0, s1):
        c = lax.axis_index("c")
        s = lax.axis_index("s")
        w = s * NC + c

        zeros = jnp.zeros((LK,), jnp.float32)

        @pl.loop(0, K)
        def _(i):
            for j in range(W // LK):
                rows0[i, pl.ds(j * LK, LK)] = zeros

        for r in range(RPT // K):
            pltpu.sync_copy(rows0, acc.at[pl.ds(s * RPT + r * K, K)])

        plsc.subcore_barrier()

        for p in range(NP):
            pltpu.sync_copy(srcq.at[w, pl.ds(p * QP, QP)], src_v)
            pltpu.sync_copy(dstq.at[w, pl.ds(p * QP, QP)], dst_v)

            pltpu.async_copy(m_hbm.at[src_v.at[0]], rows0, s0)

            @pl.loop(0, QP, step=2)
            def _(q):
                pltpu.async_copy(m_hbm.at[src_v.at[q + 1]], rows1, s1)
                pltpu.make_async_copy(m_hbm.at[src_v.at[q]], rows0, s0).wait()
                pltpu.sync_copy(rows0, acc.at[dst_v.at[q]], add=True)

                @pl.when(q + 2 < QP)
                def _():
                    pltpu.async_copy(m_hbm.at[src_v.at[q + 2]], rows0, s0)

                pltpu.make_async_copy(m_hbm.at[src_v.at[q + 1]], rows1, s1).wait()
                pltpu.sync_copy(rows1, acc.at[dst_v.at[q + 1]], add=True)

        plsc.subcore_barrier()

        for r in range(RPT // K):
            sl = pl.ds(s * RPT + r * K, K)
            pltpu.sync_copy(acc.at[sl], out.at[c, sl])

    return _agg


_agg_h = _make_agg(H)
_agg_c = _agg_h


def _tc0_body(feat_ref, w_ref, l_ref, od0, od1, m_ref, s_ref):
    x = feat_ref[...]
    ns = lax.rsqrt(jnp.maximum(od0[...] + od1[...], 1.0))
    m_ref[...] = jnp.dot(x * ns, w_ref[...], preferred_element_type=jnp.float32)
    s_ref[...] = jnp.dot(x, l_ref[...], preferred_element_type=jnp.float32)


def _col(shape):
    return pl.BlockSpec(shape, lambda i: (0,) * len(shape))


def _rowblk(w):
    return pl.BlockSpec((RB, w), lambda i: (i, 0))


_DEG_SPEC = pl.BlockSpec((RB, 1), lambda i: (i, 0))


_tc0 = pl.pallas_call(
    _tc0_body,
    grid=(N_PAD // RB,),
    in_specs=[_rowblk(D_IN), _col((D_IN, H)), _col((D_IN, H)), _DEG_SPEC, _DEG_SPEC],
    out_specs=[_rowblk(H), _rowblk(H)],
    out_shape=[
        jax.ShapeDtypeStruct((N_PAD, H), jnp.float32),
        jax.ShapeDtypeStruct((N_PAD, H), jnp.float32),
    ],
)


def _make_tc_mid(wout):
    def body(aggp, sp_ref, id0, id1, od0, od1, g_ref, be_ref, w_ref, l_ref,
             m_ref, s_ref):
        a = aggp[0] + aggp[1]
        nd = lax.rsqrt(jnp.maximum(id0[...] + id1[...], 1.0))
        h = a * nd + sp_ref[...]
        h = h * (g_ref[...] * BN_SCALE) + be_ref[...]
        h = jnp.maximum(h, 0.0)
        ns = lax.rsqrt(jnp.maximum(od0[...] + od1[...], 1.0))
        m_ref[...] = jnp.dot(h * ns, w_ref[...], preferred_element_type=jnp.float32)
        s_ref[...] = jnp.dot(h, l_ref[...], preferred_element_type=jnp.float32)

    return pl.pallas_call(
        body,
        grid=(N_PAD // RB,),
        in_specs=[
            pl.BlockSpec((NC, RB, H), lambda i: (0, i, 0)),
            _rowblk(H), _DEG_SPEC, _DEG_SPEC, _DEG_SPEC, _DEG_SPEC,
            _col((1, H)), _col((1, H)), _col((D_IN, wout)), _col((D_IN, wout)),
        ],
        out_specs=[_rowblk(wout), _rowblk(wout)],
        out_shape=[
            jax.ShapeDtypeStruct((N_PAD, wout), jnp.float32),
            jax.ShapeDtypeStruct((N_PAD, wout), jnp.float32),
        ],
    )


_tc1 = _make_tc_mid(H)
_tc2 = _make_tc_mid(C_PAD)


def _tc3_body(aggp, sp_ref, id0, id1, b_ref, out_ref):
    a = aggp[0] + aggp[1]
    nd = lax.rsqrt(jnp.maximum(id0[...] + id1[...], 1.0))
    out_ref[...] = a * nd + b_ref[...] + sp_ref[...]


_tc3 = pl.pallas_call(
    _tc3_body,
    grid=(N_PAD // RB,),
    in_specs=[
        pl.BlockSpec((NC, RB, C_PAD), lambda i: (0, i, 0)),
        _rowblk(C_PAD), _DEG_SPEC, _DEG_SPEC, _col((1, C_PAD)),
    ],
    out_specs=pl.BlockSpec((RB, C_PAD), lambda i: (i, 0)),
    out_shape=jax.ShapeDtypeStruct((N, C), jnp.float32),
)


def kernel(feat, edge_index, order_attn, W0, W1, W2, b2, L0, L1, L2,
           g0, be0, g1, be1):
    del order_attn
    fill = N + jax.lax.iota(jnp.int32, E_PAD - E) % (N_PAD - N)
    srcq = jnp.concatenate([edge_index[0], fill]).reshape(NW, QC, K)
    dstq = jnp.concatenate([edge_index[1], fill]).reshape(NW, QC, K)

    degs = _deg_kernel(srcq, dstq)
    od0 = degs[0, 0].reshape(N_PAD, 1)
    od1 = degs[1, 0].reshape(N_PAD, 1)
    id0 = degs[0, 1].reshape(N_PAD, 1)
    id1 = degs[1, 1].reshape(N_PAD, 1)

    W2p = jnp.pad(W2, ((0, 0), (0, C_PAD - C)))
    L2p = jnp.pad(L2, ((0, 0), (0, C_PAD - C)))
    b2p = jnp.pad(b2, (0, C_PAD - C)).reshape(1, C_PAD)
    g0r, be0r = g0.reshape(1, H), be0.reshape(1, H)
    g1r, be1r = g1.reshape(1, H), be1.reshape(1, H)

    m0, s0 = _tc0(feat, W0, L0, od0, od1)
    a0 = _agg_h(m0, srcq, dstq)
    m1, s1 = _tc1(a0, s0, id0, id1, od0, od1, g0r, be0r, W1, L1)
    a1 = _agg_h(m1, srcq, dstq)
    m2, s2 = _tc2(a1, s1, id0, id1, od0, od1, g1r, be1r, W2p, L2p)
    a2 = _agg_c(m2, srcq, dstq)
    return _tc3(a2, s2, id0, id1, b2p)


# speedup vs baseline: 1.0132x; 1.0088x over previous
import functools

import jax
import jax.numpy as jnp
from jax import lax
from jax.experimental import pallas as pl
from jax.experimental.pallas import tpu as pltpu
from jax.experimental.pallas import tpu_sc as plsc

N = 10000
E = 320000
D_IN = 128
H = 128
C = 40
C_PAD = 128
BN_SCALE = 1.0 / (1.0 + 1e-5) ** 0.5

NC = 2
NS = 16
NW = NC * NS
LK = 16

K = 128
QC = 80
EW = K * QC
E_PAD = EW * NW
N_PAD = 10240
RPT = N_PAD // NS
ZR = 128

RB = 1024


_SC_MESH = plsc.VectorSubcoreMesh(core_axis_name="c", subcore_axis_name="s")
_SC_PARAMS = pltpu.CompilerParams(needs_layout_passes=False)


@functools.partial(
    pl.kernel,
    out_type=jax.ShapeDtypeStruct((NC, 2, N_PAD), jnp.float32),
    mesh=_SC_MESH,
    compiler_params=_SC_PARAMS,
    scratch_types=[
        pltpu.VMEM((QC, K), jnp.int32),
        pltpu.VMEM((QC, K), jnp.int32),
        pltpu.VMEM((N_PAD,), jnp.float32),
        pltpu.VMEM((N_PAD,), jnp.float32),
        pltpu.VMEM((RPT,), jnp.float32),
        pltpu.VMEM((RPT,), jnp.float32),
        pltpu.MemorySpace.VMEM_SHARED((2, NS, N_PAD), jnp.float32),
    ],
)
def _deg_kernel(srcq, dstq, out, src_v, dst_v, deg_s, deg_d, tmp_v, acc_v, sh):
    c = lax.axis_index("c")
    s = lax.axis_index("s")
    w = s * NC + c

    pltpu.sync_copy(srcq.at[w], src_v)
    pltpu.sync_copy(dstq.at[w], dst_v)

    zeros = jnp.zeros((LK,), jnp.float32)

    @pl.loop(0, N_PAD // LK)
    def _(i):
        deg_s[pl.ds(i * LK, LK)] = zeros
        deg_d[pl.ds(i * LK, LK)] = zeros

    ones = jnp.ones((LK,), jnp.float32)

    @pl.loop(0, QC)
    def _(q):
        for g in range(K // LK):
            idx_s = src_v[q, pl.ds(g * LK, LK)]
            plsc.addupdate_scatter(deg_s, [idx_s], ones)
            idx_d = dst_v[q, pl.ds(g * LK, LK)]
            plsc.addupdate_scatter(deg_d, [idx_d], ones)

    pltpu.sync_copy(deg_s, sh.at[0, s])
    pltpu.sync_copy(deg_d, sh.at[1, s])
    plsc.subcore_barrier()

    for d in range(2):
        pltpu.sync_copy(sh.at[d, 0, pl.ds(s * RPT, RPT)], acc_v)
        for p in range(1, NS):
            pltpu.sync_copy(sh.at[d, p, pl.ds(s * RPT, RPT)], tmp_v)
            for j in range(RPT // LK):
                sl = pl.ds(j * LK, LK)
                acc_v[sl] = acc_v[sl] + tmp_v[sl]
        pltpu.sync_copy(acc_v, out.at[c, d, pl.ds(s * RPT, RPT)])


NP = 2
QP = QC // NP


def _make_agg(W):
    @functools.partial(
        pl.kernel,
        out_type=jax.ShapeDtypeStruct((NC, N_PAD, W), jnp.float32),
        mesh=_SC_MESH,
        compiler_params=_SC_PARAMS,
        scratch_types=[
            pltpu.VMEM((QP, K), jnp.int32),
            pltpu.VMEM((QP, K), jnp.int32),
            pltpu.VMEM((K, W), jnp.float32),
            pltpu.VMEM((K, W), jnp.float32),
            pltpu.MemorySpace.VMEM_SHARED((N_PAD, W), jnp.float32),
            pltpu.SemaphoreType.DMA,
            pltpu.SemaphoreType.DMA,
        ],
    )
    def _agg(m_hbm, srcq, dstq, out, src_v, dst_v, rows0, rows1, acc, s0, s1):
        c = lax.axis_index("c")
        s = lax.axis_index("s")
        w = s * NC + c

        zeros = jnp.zeros((LK,), jnp.float32)

        @pl.loop(0, K)
        def _(i):
            for j in range(W // LK):
                rows0[i, pl.ds(j * LK, LK)] = zeros

        for r in range(RPT // K):
            pltpu.sync_copy(rows0, acc.at[pl.ds(s * RPT + r * K, K)])

        plsc.subcore_barrier()

        for p in range(NP):
            pltpu.sync_copy(srcq.at[w, pl.ds(p * QP, QP)], src_v)
            pltpu.sync_copy(dstq.at[w, pl.ds(p * QP, QP)], dst_v)

            pltpu.async_copy(m_hbm.at[src_v.at[0]], rows0, s0)

            @pl.loop(0, QP, step=2)
            def _(q):
                pltpu.async_copy(m_hbm.at[src_v.at[q + 1]], rows1, s1)
                pltpu.make_async_copy(m_hbm.at[src_v.at[q]], rows0, s0).wait()
                pltpu.sync_copy(rows0, acc.at[dst_v.at[q]], add=True)

                @pl.when(q + 2 < QP)
                def _():
                    pltpu.async_copy(m_hbm.at[src_v.at[q + 2]], rows0, s0)

                pltpu.make_async_copy(m_hbm.at[src_v.at[q + 1]], rows1, s1).wait()
                pltpu.sync_copy(rows1, acc.at[dst_v.at[q + 1]], add=True)

        plsc.subcore_barrier()

        for r in range(RPT // K):
            sl = pl.ds(s * RPT + r * K, K)
            pltpu.sync_copy(acc.at[sl], out.at[c, sl])

    return _agg


_agg_h = _make_agg(H)
_agg_c = _agg_h


def _tc0_body(feat_ref, w_ref, l_ref, z_ref, s_ref):
    x = feat_ref[...]
    z_ref[...] = jnp.dot(x, w_ref[...], preferred_element_type=jnp.float32)
    s_ref[...] = jnp.dot(x, l_ref[...], preferred_element_type=jnp.float32)


def _scale_body(z_ref, od0, od1, m_ref):
    ns = lax.rsqrt(jnp.maximum(od0[...] + od1[...], 1.0))
    m_ref[...] = z_ref[...] * ns


def _col(shape):
    return pl.BlockSpec(shape, lambda i: (0,) * len(shape))


def _rowblk(w):
    return pl.BlockSpec((RB, w), lambda i: (i, 0))


_DEG_SPEC = pl.BlockSpec((RB, 1), lambda i: (i, 0))


_tc0 = pl.pallas_call(
    _tc0_body,
    grid=(N_PAD // RB,),
    in_specs=[_rowblk(D_IN), _col((D_IN, H)), _col((D_IN, H))],
    out_specs=[_rowblk(H), _rowblk(H)],
    out_shape=[
        jax.ShapeDtypeStruct((N_PAD, H), jnp.float32),
        jax.ShapeDtypeStruct((N_PAD, H), jnp.float32),
    ],
)

_scale = pl.pallas_call(
    _scale_body,
    grid=(N_PAD // RB,),
    in_specs=[_rowblk(H), _DEG_SPEC, _DEG_SPEC],
    out_specs=_rowblk(H),
    out_shape=jax.ShapeDtypeStruct((N_PAD, H), jnp.float32),
)


def _make_tc_mid(wout):
    def body(aggp, sp_ref, id0, id1, od0, od1, g_ref, be_ref, w_ref, l_ref,
             m_ref, s_ref):
        a = aggp[0] + aggp[1]
        nd = lax.rsqrt(jnp.maximum(id0[...] + id1[...], 1.0))
        h = a * nd + sp_ref[...]
        h = h * (g_ref[...] * BN_SCALE) + be_ref[...]
        h = jnp.maximum(h, 0.0)
        ns = lax.rsqrt(jnp.maximum(od0[...] + od1[...], 1.0))
        m_ref[...] = jnp.dot(h * ns, w_ref[...], preferred_element_type=jnp.float32)
        s_ref[...] = jnp.dot(h, l_ref[...], preferred_element_type=jnp.float32)

    return pl.pallas_call(
        body,
        grid=(N_PAD // RB,),
        in_specs=[
            pl.BlockSpec((NC, RB, H), lambda i: (0, i, 0)),
            _rowblk(H), _DEG_SPEC, _DEG_SPEC, _DEG_SPEC, _DEG_SPEC,
            _col((1, H)), _col((1, H)), _col((D_IN, wout)), _col((D_IN, wout)),
        ],
        out_specs=[_rowblk(wout), _rowblk(wout)],
        out_shape=[
            jax.ShapeDtypeStruct((N_PAD, wout), jnp.float32),
            jax.ShapeDtypeStruct((N_PAD, wout), jnp.float32),
        ],
    )


_tc1 = _make_tc_mid(H)
_tc2 = _make_tc_mid(C_PAD)


def _tc3_body(aggp, sp_ref, id0, id1, b_ref, out_ref):
    a = aggp[0] + aggp[1]
    nd = lax.rsqrt(jnp.maximum(id0[...] + id1[...], 1.0))
    out_ref[...] = a * nd + b_ref[...] + sp_ref[...]


_tc3 = pl.pallas_call(
    _tc3_body,
    grid=(N_PAD // RB,),
    in_specs=[
        pl.BlockSpec((NC, RB, C_PAD), lambda i: (0, i, 0)),
        _rowblk(C_PAD), _DEG_SPEC, _DEG_SPEC, _col((1, C_PAD)),
    ],
    out_specs=pl.BlockSpec((RB, C_PAD), lambda i: (i, 0)),
    out_shape=jax.ShapeDtypeStruct((N, C), jnp.float32),
)


def kernel(feat, edge_index, order_attn, W0, W1, W2, b2, L0, L1, L2,
           g0, be0, g1, be1):
    del order_attn
    fill = N + jax.lax.iota(jnp.int32, E_PAD - E) % (N_PAD - N)
    srcq = jnp.concatenate([edge_index[0], fill]).reshape(NW, QC, K)
    dstq = jnp.concatenate([edge_index[1], fill]).reshape(NW, QC, K)

    degs = _deg_kernel(srcq, dstq)
    od0 = degs[0, 0].reshape(N_PAD, 1)
    od1 = degs[1, 0].reshape(N_PAD, 1)
    id0 = degs[0, 1].reshape(N_PAD, 1)
    id1 = degs[1, 1].reshape(N_PAD, 1)

    W2p = jnp.pad(W2, ((0, 0), (0, C_PAD - C)))
    L2p = jnp.pad(L2, ((0, 0), (0, C_PAD - C)))
    b2p = jnp.pad(b2, (0, C_PAD - C)).reshape(1, C_PAD)
    g0r, be0r = g0.reshape(1, H), be0.reshape(1, H)
    g1r, be1r = g1.reshape(1, H), be1.reshape(1, H)

    z0, s0 = _tc0(feat, W0, L0)
    m0 = _scale(z0, od0, od1)
    a0 = _agg_h(m0, srcq, dstq)
    m1, s1 = _tc1(a0, s0, id0, id1, od0, od1, g0r, be0r, W1, L1)
    a1 = _agg_h(m1, srcq, dstq)
    m2, s2 = _tc2(a1, s1, id0, id1, od0, od1, g1r, be1r, W2p, L2p)
    a2 = _agg_c(m2, srcq, dstq)
    return _tc3(a2, s2, id0, id1, b2p)
